# baseline (speedup 1.0000x reference)
import functools

import jax
import jax.numpy as jnp
from jax import lax
from jax.experimental import pallas as pl
from jax.experimental.pallas import tpu as pltpu
from jax.experimental.pallas import tpu_sc as plsc

VOCAB = 1000000
EMBED_DIM = 64
LANES = 128
SBLK = 128


@functools.partial(jax.jit, static_argnums=(2, 3, 4))
def _sc_gather(ids_flat, tbl2, n_workers, n_sent, n_words):
    mesh = plsc.VectorSubcoreMesh(core_axis_name="c", subcore_axis_name="s")
    num_cores = plsc.get_sparse_core_info().num_cores
    ids_per_w = SBLK * n_words
    n_sblk = n_sent // SBLK

    @functools.partial(
        pl.kernel,
        mesh=mesh,
        compiler_params=pltpu.CompilerParams(
            use_tc_tiling_on_sc=False, needs_layout_passes=False
        ),
        out_type=jax.ShapeDtypeStruct(
            (n_words, EMBED_DIM // 8, n_sblk, 8, SBLK), jnp.float32
        ),
        scratch_types=[
            pltpu.VMEM((ids_per_w,), jnp.int32),
            pltpu.VMEM((SBLK,), jnp.int32),
            pltpu.VMEM((SBLK, EMBED_DIM), jnp.float32),
            pltpu.VMEM((EMBED_DIM // 8, 8, SBLK), jnp.float32),
            pltpu.SemaphoreType.DMA,
        ],
    )
    def k(ids_hbm, tbl_hbm, out_hbm, slab_v, idx_v, rows_v, tr_v, sem):
        wid = lax.axis_index("s") * num_cores + lax.axis_index("c")
        pltpu.sync_copy(ids_hbm.at[pl.ds(wid * ids_per_w, ids_per_w)], slab_v)
        iota = lax.iota(jnp.int32, 16)
        rowsel = [iota + 16 * j for j in range(8)]

        def word(w, carry):
            for kk in range(8):
                ids16 = plsc.load_gather(slab_v, [rowsel[kk] * n_words + w])
                idx_v[pl.ds(16 * kk, 16)] = ids16 * 2
            pltpu.async_copy(tbl_hbm.at[idx_v], rows_v, sem).wait()

            def dim(d, c2):
                dh = d // 8
                dl = d - 8 * dh
                col = jnp.broadcast_to(d, (16,))
                for j in range(8):
                    val = plsc.load_gather(rows_v, [rowsel[j], col])
                    tr_v[dh, dl, pl.ds(16 * j, 16)] = val
                return c2

            lax.fori_loop(0, EMBED_DIM, dim, 0, unroll=2)
            pltpu.sync_copy(tr_v, out_hbm.at[w, :, wid])
            return carry

        lax.fori_loop(0, n_words, word, 0)

    return k(ids_flat, tbl2)


def kernel(input_ids, table):
    S, W = input_ids.shape
    info = plsc.get_sparse_core_info()
    n_workers = info.num_cores * info.num_subcores
    tbl2 = jnp.pad(table, ((0, 0), (0, LANES - EMBED_DIM)))
    tbl2 = tbl2.reshape(2 * VOCAB, EMBED_DIM)
    out5 = _sc_gather(input_ids.reshape(S * W), tbl2, n_workers, S, W)
    return out5.transpose(2, 4, 0, 1, 3).reshape(S, W, EMBED_DIM)


# speedup vs baseline: 1.0794x; 1.0794x over previous
import functools

import jax
import jax.numpy as jnp
from jax import lax
from jax.experimental import pallas as pl
from jax.experimental.pallas import tpu as pltpu
from jax.experimental.pallas import tpu_sc as plsc

VOCAB = 1000000
EMBED_DIM = 64
LANES = 128
SBLK = 128


@functools.partial(jax.jit, static_argnums=(2, 3, 4))
def _sc_gather(ids_flat, tbl2, n_workers, n_sent, n_words):
    mesh = plsc.VectorSubcoreMesh(core_axis_name="c", subcore_axis_name="s")
    num_cores = plsc.get_sparse_core_info().num_cores
    ids_per_w = SBLK * n_words
    n_sblk = n_sent // SBLK

    @functools.partial(
        pl.kernel,
        mesh=mesh,
        compiler_params=pltpu.CompilerParams(
            use_tc_tiling_on_sc=False, needs_layout_passes=False
        ),
        out_type=jax.ShapeDtypeStruct(
            (n_words, EMBED_DIM // 8, n_sblk, 8, SBLK), jnp.float32
        ),
        scratch_types=[
            pltpu.VMEM((ids_per_w,), jnp.int32),
            pltpu.VMEM((2, SBLK), jnp.int32),
            pltpu.VMEM((2, SBLK, EMBED_DIM), jnp.float32),
            pltpu.VMEM((2, EMBED_DIM // 8, 8, SBLK), jnp.float32),
            pltpu.SemaphoreType.DMA,
            pltpu.SemaphoreType.DMA,
            pltpu.SemaphoreType.DMA,
            pltpu.SemaphoreType.DMA,
        ],
    )
    def k(ids_hbm, tbl_hbm, out_hbm, slab_v, idx_v, rows_v, tr_v, g0, g1, o0, o1):
        wid = lax.axis_index("s") * num_cores + lax.axis_index("c")
        pltpu.sync_copy(ids_hbm.at[pl.ds(wid * ids_per_w, ids_per_w)], slab_v)
        iota = lax.iota(jnp.int32, 16)
        rowsel = [iota + 16 * j for j in range(8)]
        gsem = [g0, g1]
        osem = [o0, o1]

        def extract_fire(w, b):
            for kk in range(8):
                ids16 = plsc.load_gather(slab_v, [rowsel[kk] * n_words + w])
                idx_v[b, pl.ds(16 * kk, 16)] = ids16 * 2
            pltpu.async_copy(tbl_hbm.at[idx_v.at[b]], rows_v.at[b], gsem[b])

        def wait_gather(b):
            pltpu.make_async_copy(
                tbl_hbm.at[idx_v.at[b]], rows_v.at[b], gsem[b]
            ).wait()

        def wait_out(b, w):
            pltpu.make_async_copy(
                tr_v.at[b], out_hbm.at[w, :, wid], osem[b]
            ).wait()

        def transpose(b):
            rows_b = rows_v.at[b]
            tr_b = tr_v.at[b]

            def dim(d, c2):
                dh = d // 8
                dl = d - 8 * dh
                col = jnp.broadcast_to(d, (16,))
                for j in range(8):
                    val = plsc.load_gather(rows_b, [rowsel[j], col])
                    tr_b[dh, dl, pl.ds(16 * j, 16)] = val
                return c2

            lax.fori_loop(0, EMBED_DIM, dim, 0, unroll=4)

        extract_fire(0, 0)

        def pair(t, carry):
            w0 = 2 * t
            extract_fire(w0 + 1, 1)
            wait_gather(0)

            @pl.when(t > 0)
            def _():
                wait_out(0, w0)

            transpose(0)
            pltpu.async_copy(tr_v.at[0], out_hbm.at[w0, :, wid], osem[0])

            @pl.when(w0 + 2 < n_words)
            def _():
                extract_fire(w0 + 2, 0)

            wait_gather(1)

            @pl.when(t > 0)
            def _():
                wait_out(1, w0 + 1)

            transpose(1)
            pltpu.async_copy(tr_v.at[1], out_hbm.at[w0 + 1, :, wid], osem[1])
            return carry

        lax.fori_loop(0, n_words // 2, pair, 0)
        wait_out(0, 0)
        wait_out(1, 0)

    return k(ids_flat, tbl2)


def kernel(input_ids, table):
    S, W = input_ids.shape
    info = plsc.get_sparse_core_info()
    n_workers = info.num_cores * info.num_subcores
    tbl2 = jnp.pad(table, ((0, 0), (0, LANES - EMBED_DIM)))
    tbl2 = tbl2.reshape(2 * VOCAB, EMBED_DIM)
    out5 = _sc_gather(input_ids.reshape(S * W), tbl2, n_workers, S, W)
    return out5.transpose(2, 4, 0, 1, 3).reshape(S, W, EMBED_DIM)


# speedup vs baseline: 1.2966x; 1.2012x over previous
import functools

import jax
import jax.numpy as jnp
from jax import lax
from jax.experimental import pallas as pl
from jax.experimental.pallas import tpu as pltpu
from jax.experimental.pallas import tpu_sc as plsc

VOCAB = 1000000
EMBED_DIM = 64
LANES = 128
SBLK = 128


@functools.partial(jax.jit, static_argnums=(2, 3, 4))
def _sc_gather(ids_flat, tbl2, n_workers, n_sent, n_words):
    mesh = plsc.VectorSubcoreMesh(core_axis_name="c", subcore_axis_name="s")
    num_cores = plsc.get_sparse_core_info().num_cores
    ids_per_w = SBLK * n_words
    n_sblk = n_sent // SBLK

    @functools.partial(
        pl.kernel,
        mesh=mesh,
        compiler_params=pltpu.CompilerParams(
            use_tc_tiling_on_sc=False, needs_layout_passes=False
        ),
        out_type=jax.ShapeDtypeStruct(
            (n_words, EMBED_DIM // 8, n_sblk, 8, SBLK), jnp.float32
        ),
        scratch_types=[
            pltpu.VMEM((ids_per_w,), jnp.int32),
            pltpu.VMEM((2, SBLK), jnp.int32),
            pltpu.VMEM((2, SBLK, EMBED_DIM), jnp.float32),
            pltpu.VMEM((2, EMBED_DIM // 8, 8, SBLK), jnp.float32),
            pltpu.SemaphoreType.DMA,
            pltpu.SemaphoreType.DMA,
            pltpu.SemaphoreType.DMA,
            pltpu.SemaphoreType.DMA,
        ],
    )
    def k(ids_hbm, tbl_hbm, out_hbm, slab_v, idx_v, rows_v, tr_v, g0, g1, o0, o1):
        wid = lax.axis_index("s") * num_cores + lax.axis_index("c")
        pltpu.sync_copy(ids_hbm.at[pl.ds(wid * ids_per_w, ids_per_w)], slab_v)
        iota = lax.iota(jnp.int32, 16)
        rowsel = [iota + 16 * j for j in range(8)]
        gsem = [g0, g1]
        osem = [o0, o1]

        def extract_fire(w, b):
            for kk in range(8):
                ids16 = plsc.load_gather(slab_v, [rowsel[kk] * n_words + w])
                idx_v[b, pl.ds(16 * kk, 16)] = ids16 * 2
            pltpu.async_copy(tbl_hbm.at[idx_v.at[b]], rows_v.at[b], gsem[b])

        def wait_gather(b):
            pltpu.make_async_copy(
                tbl_hbm.at[idx_v.at[b]], rows_v.at[b], gsem[b]
            ).wait()

        def wait_out(b, w):
            pltpu.make_async_copy(
                tr_v.at[b], out_hbm.at[w, :, wid], osem[b]
            ).wait()

        def transpose(b):
            rows_b = rows_v.at[b]
            tr_b = tr_v.at[b]

            @plsc.parallel_loop(0, EMBED_DIM, unroll=4)
            def dim(d):
                dh = d // 8
                dl = d - 8 * dh
                col = jnp.broadcast_to(d, (16,))
                for j in range(8):
                    val = plsc.load_gather(rows_b, [rowsel[j], col])
                    tr_b[dh, dl, pl.ds(16 * j, 16)] = val

        extract_fire(0, 0)

        def pair(t, carry):
            w0 = 2 * t
            extract_fire(w0 + 1, 1)
            wait_gather(0)

            @pl.when(t > 0)
            def _():
                wait_out(0, w0)

            transpose(0)
            pltpu.async_copy(tr_v.at[0], out_hbm.at[w0, :, wid], osem[0])

            @pl.when(w0 + 2 < n_words)
            def _():
                extract_fire(w0 + 2, 0)

            wait_gather(1)

            @pl.when(t > 0)
            def _():
                wait_out(1, w0 + 1)

            transpose(1)
            pltpu.async_copy(tr_v.at[1], out_hbm.at[w0 + 1, :, wid], osem[1])
            return carry

        lax.fori_loop(0, n_words // 2, pair, 0)
        wait_out(0, 0)
        wait_out(1, 0)

    return k(ids_flat, tbl2)


def kernel(input_ids, table):
    S, W = input_ids.shape
    info = plsc.get_sparse_core_info()
    n_workers = info.num_cores * info.num_subcores
    tbl2 = jnp.pad(table, ((0, 0), (0, LANES - EMBED_DIM)))
    tbl2 = tbl2.reshape(2 * VOCAB, EMBED_DIM)
    out5 = _sc_gather(input_ids.reshape(S * W), tbl2, n_workers, S, W)
    return out5.transpose(2, 4, 0, 1, 3).reshape(S, W, EMBED_DIM)
